# baseline (speedup 1.0000x reference)
import jax
import jax.numpy as jnp
from jax.experimental import pallas as pl
from jax.experimental.pallas import tpu as pltpu

_DIM = 4096
_NE = 64
_TOPK = 8
_EPS = 1e-06
_CAP = 3.0

_NS = 4
_BT = 256
_BTO = 4096


def _gate_kernel(x0_ref, x1_ref, x2_ref, x3_ref, wt_ref, b_ref, out_ref,
                 ms_ref, lg_ref, d_ref):
    i = pl.program_id(0)
    n = ms_ref.shape[0]
    span = n // _NS
    nb = span // _BT

    @pl.when(i <= nb)
    def _phase_a():
        logits = lg_ref[...]

        wt = wt_ref[...]
        new_logits = jnp.concatenate(
            [jnp.dot(xr[...], wt,
                     preferred_element_type=jnp.float32,
                     precision=jax.lax.Precision.DEFAULT)
             for xr in (x0_ref, x1_ref, x2_ref, x3_ref)],
            axis=0,
        ) + b_ref[...]
        lg_ref[...] = new_logits

        col = jax.lax.broadcasted_iota(jnp.int32, logits.shape, 1).astype(
            jnp.float32)
        work = logits
        mask = jnp.zeros(logits.shape, dtype=jnp.bool_)
        m = None
        for r in range(_TOPK):
            cur = jnp.max(work, axis=-1, keepdims=True)
            if r == 0:
                m = cur
            cand = jnp.where(work == cur, col, 64.0)
            sel = jnp.min(cand, axis=-1, keepdims=True)
            hit = col == sel
            mask = jnp.logical_or(mask, hit)
            work = jnp.where(hit, -3.4e38, work)
        e = jnp.exp(logits - m)
        s = jnp.sum(e, axis=-1, keepdims=True)
        masked = jnp.where(mask, e, 0.0) / s
        masked = jnp.where(i > 0, masked, 0.0)

        base = jnp.maximum(i - 1, 0) * _BT
        for st in range(_NS):
            ms_ref[pl.ds(base + st * span, _BT), :] = (
                masked[st * _BT:(st + 1) * _BT, :])

        contrib = jnp.sum(masked, axis=0, keepdims=True)
        d_ref[...] = jnp.where(i > 1, d_ref[...], 0.0) + contrib

    @pl.when(i > nb)
    def _phase_b():
        j = i - (nb + 1)
        scale = _CAP / (d_ref[...] + _EPS)
        out_ref[...] = ms_ref[pl.ds(j * _BTO, _BTO), :] * scale


def kernel(x, W, b):
    n = x.shape[0]
    span = n // _NS
    nb = span // _BT
    nbo = n // _BTO
    wt = W.T
    b2 = b.reshape(1, _NE)

    def _xspec(st):
        return pl.BlockSpec(
            (_BT, _DIM),
            lambda i, st=st: (jnp.minimum(i, nb - 1) + st * nb, 0))

    return pl.pallas_call(
        _gate_kernel,
        grid=(nb + 1 + nbo,),
        in_specs=[
            _xspec(0), _xspec(1), _xspec(2), _xspec(3),
            pl.BlockSpec((_DIM, _NE), lambda i: (0, 0)),
            pl.BlockSpec((1, _NE), lambda i: (0, 0)),
        ],
        out_specs=pl.BlockSpec(
            (_BTO, _NE), lambda i: (jnp.maximum(i - (nb + 1), 0), 0)),
        out_shape=jax.ShapeDtypeStruct((n, _NE), jnp.float32),
        scratch_shapes=[
            pltpu.VMEM((n, _NE), jnp.float32),
            pltpu.VMEM((_NS * _BT, _NE), jnp.float32),
            pltpu.VMEM((1, _NE), jnp.float32),
        ],
    )(x, x, x, x, wt, b2)


# speedup vs baseline: 1.0014x; 1.0014x over previous
import jax
import jax.numpy as jnp
from jax.experimental import pallas as pl
from jax.experimental.pallas import tpu as pltpu

_DIM = 4096
_NE = 64
_TOPK = 8
_EPS = 1e-06
_CAP = 3.0

_NS = 2
_BT = 512
_BTO = 4096


def _gate_kernel(x0_ref, x1_ref, wt_ref, b_ref, out_ref, ms_ref, lg_ref,
                 d_ref):
    i = pl.program_id(0)
    span = ms_ref.shape[0] // _NS
    nb = span // _BT

    @pl.when(i <= nb)
    def _phase_a():
        logits = lg_ref[...]

        wt = wt_ref[...]
        new_logits = jnp.concatenate(
            [jnp.dot(xr[...], wt,
                     preferred_element_type=jnp.float32,
                     precision=jax.lax.Precision.DEFAULT)
             for xr in (x0_ref, x1_ref)],
            axis=0,
        ) + b_ref[...]
        lg_ref[...] = new_logits

        col = jax.lax.broadcasted_iota(jnp.int32, logits.shape, 1).astype(
            jnp.float32)
        work = logits
        mask = jnp.zeros(logits.shape, dtype=jnp.bool_)
        m = None
        for r in range(_TOPK):
            cur = jnp.max(work, axis=-1, keepdims=True)
            if r == 0:
                m = cur
            cand = jnp.where(work == cur, col, 64.0)
            sel = jnp.min(cand, axis=-1, keepdims=True)
            hit = col == sel
            mask = jnp.logical_or(mask, hit)
            work = jnp.where(hit, -3.4e38, work)
        e = jnp.exp(logits - m)
        s = jnp.sum(e, axis=-1, keepdims=True)
        masked = jnp.where(mask, e, 0.0) / s
        masked = jnp.where(i > 0, masked, 0.0)

        base = jnp.maximum(i - 1, 0) * _BT
        for st in range(_NS):
            ms_ref[pl.ds(base + st * span, _BT), :] = (
                masked[st * _BT:(st + 1) * _BT, :])

        contrib = jnp.sum(masked, axis=0, keepdims=True)
        d_ref[...] = jnp.where(i > 1, d_ref[...], 0.0) + contrib

    @pl.when(i > nb)
    def _phase_b():
        j = i - (nb + 1)
        scale = _CAP / (d_ref[...] + _EPS)
        out_ref[...] = ms_ref[pl.ds(j * _BTO, _BTO), :] * scale


def kernel(x, W, b):
    n = x.shape[0]
    span = n // _NS
    nb = span // _BT
    wt = W.T
    b2 = b.reshape(1, _NE)
    nbo = n // _BTO

    def _xspec(st):
        return pl.BlockSpec(
            (_BT, _DIM),
            lambda i, st=st: (jnp.minimum(i, nb - 1) + st * nb, 0))

    return pl.pallas_call(
        _gate_kernel,
        grid=(nb + 1 + nbo,),
        in_specs=[
            _xspec(0), _xspec(1),
            pl.BlockSpec((_DIM, _NE), lambda i: (0, 0)),
            pl.BlockSpec((1, _NE), lambda i: (0, 0)),
        ],
        out_specs=pl.BlockSpec(
            (_BTO, _NE), lambda i: (jnp.maximum(i - (nb + 1), 0), 0)),
        out_shape=jax.ShapeDtypeStruct((n, _NE), jnp.float32),
        scratch_shapes=[
            pltpu.VMEM((n, _NE), jnp.float32),
            pltpu.VMEM((_NS * _BT, _NE), jnp.float32),
            pltpu.VMEM((1, _NE), jnp.float32),
        ],
    )(x, x, wt, b2)


# speedup vs baseline: 1.0041x; 1.0027x over previous
import jax
import jax.numpy as jnp
from jax.experimental import pallas as pl
from jax.experimental.pallas import tpu as pltpu

_DIM = 4096
_NE = 64
_TOPK = 8
_EPS = 1e-06
_CAP = 3.0

_BT = 1024
_BTO = 4096


def _gate_kernel(x_ref, wt_ref, b_ref, out_ref, ms_ref, lg_ref, d_ref):
    i = pl.program_id(0)
    nb = ms_ref.shape[0] // _BT

    @pl.when(i <= nb)
    def _phase_a():
        logits = lg_ref[...]

        new_logits = jnp.dot(
            x_ref[...], wt_ref[...],
            preferred_element_type=jnp.float32,
            precision=jax.lax.Precision.DEFAULT,
        ) + b_ref[...]
        lg_ref[...] = new_logits

        col = jax.lax.broadcasted_iota(jnp.int32, logits.shape, 1).astype(
            jnp.float32)
        work = logits
        mask = jnp.zeros(logits.shape, dtype=jnp.bool_)
        m = None
        for r in range(_TOPK):
            cur = jnp.max(work, axis=-1, keepdims=True)
            if r == 0:
                m = cur
            cand = jnp.where(work == cur, col, 64.0)
            sel = jnp.min(cand, axis=-1, keepdims=True)
            hit = col == sel
            mask = jnp.logical_or(mask, hit)
            work = jnp.where(hit, -3.4e38, work)
        e = jnp.exp(logits - m)
        s = jnp.sum(e, axis=-1, keepdims=True)
        masked = jnp.where(mask, e, 0.0) / s
        masked = jnp.where(i > 0, masked, 0.0)

        base = jnp.maximum(i - 1, 0) * _BT
        ms_ref[pl.ds(base, _BT), :] = masked

        contrib = jnp.sum(masked, axis=0, keepdims=True)
        d_ref[...] = jnp.where(i > 1, d_ref[...], 0.0) + contrib

    @pl.when(i > nb)
    def _phase_b():
        j = i - (nb + 1)
        scale = _CAP / (d_ref[...] + _EPS)
        out_ref[...] = ms_ref[pl.ds(j * _BTO, _BTO), :] * scale


def kernel(x, W, b):
    n = x.shape[0]
    nb = n // _BT
    wt = W.T
    b2 = b.reshape(1, _NE)
    nbo = n // _BTO
    return pl.pallas_call(
        _gate_kernel,
        grid=(nb + 1 + nbo,),
        in_specs=[
            pl.BlockSpec((_BT, _DIM), lambda i: (jnp.minimum(i, nb - 1), 0)),
            pl.BlockSpec((_DIM, _NE), lambda i: (0, 0)),
            pl.BlockSpec((1, _NE), lambda i: (0, 0)),
        ],
        out_specs=pl.BlockSpec(
            (_BTO, _NE), lambda i: (jnp.maximum(i - (nb + 1), 0), 0)),
        out_shape=jax.ShapeDtypeStruct((n, _NE), jnp.float32),
        scratch_shapes=[
            pltpu.VMEM((n, _NE), jnp.float32),
            pltpu.VMEM((_BT, _NE), jnp.float32),
            pltpu.VMEM((1, _NE), jnp.float32),
        ],
    )(x, wt, b2)
